# Initial kernel scaffold; baseline (speedup 1.0000x reference)
import jax
import jax.numpy as jnp
from jax.experimental import pallas as pl


def kernel(x, edge_index, res_n_id, W, b):
    raise NotImplementedError("write your pallas kernel here")


# speedup vs baseline: 10.3108x; 10.3108x over previous
import jax
import jax.numpy as jnp
from jax import lax
from jax.experimental import pallas as pl
from jax.experimental.pallas import tpu as pltpu
from jax.experimental.pallas import tpu_sc as plsc

N_NODES = 10000
N_EDGES = 320000
ND = 5000
ND_P = 5120
D = 128
NC, NS = 2, 16
NW = NC * NS
CH = 128
NCH = 80
E_P = NW * NCH * CH
EPS = E_P // NS
RPT = ND_P // NS
RSUB = 16
XR_PT = ND_P // NW
HCH = 2048
CHX = 80


def _rsqrt16(d):
    i = plsc.bitcast(d, jnp.int32)
    i = jnp.int32(0x5F3759DF) - lax.shift_right_logical(i, 1)
    y = plsc.bitcast(i, jnp.float32)
    for _ in range(3):
        y = y * (1.5 - 0.5 * d * y * y)
    return y


def _sc_body(x_hbm, src5_hbm, dst5_hbm, rid_hbm,
             agg_out, histd_out, xr_out, xsb_out,
             hidx_v, hsrc_v, hdst_v, s5_v, d5_v,
             degs_v, degd_v, a_v, hred_v, rows_v, sbuf_v, rid_v,
             shist_s, agg_s, sem):
    c = lax.axis_index("c")
    s = lax.axis_index("s")
    w = s * NC + c

    ones = jnp.full((16,), 1.0, jnp.float32)
    zeros = jnp.zeros((16,), jnp.float32)

    def zero_hist(i, _):
        hsrc_v[pl.ds(i * 16, 16)] = zeros
        hsrc_v[pl.ds(ND_P + i * 16, 16)] = zeros
        hdst_v[pl.ds(i * 16, 16)] = zeros
        return 0
    lax.fori_loop(0, ND_P // 16, zero_hist, 0)

    def hist_pass(idx_hbm, hist_ref):
        for cc in range(NC):
            wb = s * NC + cc

            def outer(h, _):
                pltpu.sync_copy(idx_hbm.at[wb, pl.ds(h * 16, 16)], hidx_v)

                def inner(r, _):
                    for g in range(CH // 16):
                        si = hidx_v[r, pl.ds(g * 16, 16)]
                        plsc.addupdate_scatter(hist_ref, [si], ones)
                    return 0
                lax.fori_loop(0, 16, inner, 0)
                return 0
            lax.fori_loop(0, NCH // 16, outer, 0)

    hist_pass(src5_hbm, hsrc_v)
    hist_pass(dst5_hbm, hdst_v)

    def fold_body(k, _):
        sl = pl.ds(k * 16, 16)
        hsrc_v[sl] = hsrc_v[sl] + hsrc_v[pl.ds(ND_P + k * 16, 16)]
        return 0
    lax.fori_loop(0, ND_P // 16, fold_body, 0)

    pltpu.sync_copy(hsrc_v.at[pl.ds(0, ND_P)],
                    shist_s.at[pl.ds(s * ND_P, ND_P)])
    pltpu.sync_copy(hdst_v, shist_s.at[pl.ds((NS + s) * ND_P, ND_P)])
    plsc.subcore_barrier()

    r0 = s * RPT

    def reduce_hist(which, out_ref):
        for t in range(NS):
            pltpu.sync_copy(
                shist_s.at[pl.ds((which * NS + t) * ND_P + r0, RPT)],
                hred_v.at[pl.ds(t * RPT, RPT)])

        def red_body(k, _):
            acc = hred_v[pl.ds(k * 16, 16)]
            for t in range(1, NS):
                acc = acc + hred_v[pl.ds(t * RPT + k * 16, 16)]
            out_ref[pl.ds(k * 16, 16)] = acc
            return 0
        lax.fori_loop(0, RPT // 16, red_body, 0)

    reduce_hist(0, degs_v)
    reduce_hist(1, degd_v)

    def newton_body(k, _):
        d = degs_v[pl.ds(k * 16, 16)] + 1.0
        a_v[pl.ds(k * 16, 16)] = _rsqrt16(d)
        return 0
    lax.fori_loop(0, RPT // 16, newton_body, 0)

    @pl.when(c == 0)
    def _():
        pltpu.sync_copy(degd_v, histd_out.at[pl.ds(r0, RPT)])

    def xs_body(j, _):
        rr = r0 + j * RSUB
        pltpu.sync_copy(x_hbm.at[pl.ds(rr, RSUB)], sbuf_v)
        a16 = a_v[pl.ds(j * RSUB, RSUB)]
        for r in range(RSUB):
            av = a16[r]
            for cc in range(D // 16):
                sl = pl.ds(cc * 16, 16)
                sbuf_v[r, sl] = sbuf_v[r, sl] * av
        pltpu.sync_copy(sbuf_v, xsb_out.at[pl.ds(c * ND_P + rr, RSUB)])
        return 0
    lax.fori_loop(0, RPT // RSUB, xs_body, 0)

    def zero_rows(r, _):
        for cc in range(D // 16):
            rows_v[r, pl.ds(cc * 16, 16)] = zeros
        return 0
    lax.fori_loop(0, CH, zero_rows, 0)
    for k in range(RPT // CHX):
        pltpu.sync_copy(rows_v.at[pl.ds(0, CHX)],
                        agg_s.at[pl.ds(r0 + k * CHX, CHX)])
    plsc.subcore_barrier()

    pltpu.sync_copy(src5_hbm.at[w], s5_v)
    pltpu.sync_copy(dst5_hbm.at[w], d5_v)

    def edge_body(j, _):
        pltpu.async_copy(xsb_out.at[s5_v.at[j]], rows_v, sem).wait()
        pltpu.sync_copy(rows_v, agg_s.at[d5_v.at[j]], add=True)
        return 0
    lax.fori_loop(0, NCH, edge_body, 0)
    plsc.subcore_barrier()

    for k in range(RPT // CHX):
        rr = r0 + k * CHX
        pltpu.sync_copy(agg_s.at[pl.ds(rr, CHX)], rows_v.at[pl.ds(0, CHX)])
        pltpu.sync_copy(rows_v.at[pl.ds(0, CHX)],
                        agg_out.at[pl.ds(c * ND_P + rr, CHX)])

    for k in range(XR_PT // CHX):
        rr = w * XR_PT + k * CHX
        pltpu.sync_copy(rid_hbm.at[pl.ds(rr, CHX)], rid_v)
        pltpu.async_copy(x_hbm.at[rid_v], rows_v.at[pl.ds(0, CHX)],
                         sem).wait()
        pltpu.sync_copy(rows_v.at[pl.ds(0, CHX)], xr_out.at[pl.ds(rr, CHX)])


@jax.jit
def _sc_stage(x, src5, dst5, ridp):
    mesh = plsc.VectorSubcoreMesh(core_axis_name="c", subcore_axis_name="s")
    f = pl.kernel(
        _sc_body,
        out_type=[
            jax.ShapeDtypeStruct((NC * ND_P, D), jnp.float32),
            jax.ShapeDtypeStruct((ND_P,), jnp.float32),
            jax.ShapeDtypeStruct((ND_P, D), jnp.float32),
            jax.ShapeDtypeStruct((NC * ND_P, D), jnp.float32),
        ],
        mesh=mesh,
        scratch_types=[
            pltpu.VMEM((16, CH), jnp.int32),
            pltpu.VMEM((2 * ND_P,), jnp.float32),
            pltpu.VMEM((ND_P,), jnp.float32),
            pltpu.VMEM((NCH, CH), jnp.int32),
            pltpu.VMEM((NCH, CH), jnp.int32),
            pltpu.VMEM((RPT,), jnp.float32),
            pltpu.VMEM((RPT,), jnp.float32),
            pltpu.VMEM((RPT,), jnp.float32),
            pltpu.VMEM((NS * RPT,), jnp.float32),
            pltpu.VMEM((CH, D), jnp.float32),
            pltpu.VMEM((RSUB, D), jnp.float32),
            pltpu.VMEM((CHX,), jnp.int32),
            pltpu.VMEM_SHARED((2 * NS * ND_P,), jnp.float32),
            pltpu.VMEM_SHARED((ND_P, D), jnp.float32),
            pltpu.SemaphoreType.DMA,
        ],
        compiler_params=pltpu.CompilerParams(needs_layout_passes=False),
    )
    return f(x, src5, dst5, ridp)


def _tc_body(agg_ref, hist_ref, xr_ref, w_ref, b_ref, o_ref):
    deg = hist_ref[...] + 1.0
    agg = agg_ref[0] + agg_ref[1]
    z = agg * lax.rsqrt(deg) + xr_ref[...] * (1.0 / deg)
    out = jnp.dot(z, w_ref[...], preferred_element_type=jnp.float32)
    out = out + b_ref[...]
    m = jnp.max(out, axis=1, keepdims=True)
    ex = jnp.exp(out - m)
    lse = jnp.log(jnp.sum(ex, axis=1, keepdims=True)) + m
    o_ref[...] = out - lse


BR = 1000


@jax.jit
def _tc_stage(agg, histd, xr, W, b2):
    return pl.pallas_call(
        _tc_body,
        grid=(ND // BR,),
        in_specs=[
            pl.BlockSpec((NC, BR, D), lambda i: (0, i, 0)),
            pl.BlockSpec((BR, 1), lambda i: (i, 0)),
            pl.BlockSpec((BR, D), lambda i: (i, 0)),
            pl.BlockSpec((D, D), lambda i: (0, 0)),
            pl.BlockSpec((1, D), lambda i: (0, 0)),
        ],
        out_specs=pl.BlockSpec((BR, D), lambda i: (i, 0)),
        out_shape=jax.ShapeDtypeStruct((ND, D), jnp.float32),
    )(agg, histd, xr, W, b2)


def kernel(x, edge_index, res_n_id, W, b):
    pad = jnp.full((E_P - N_EDGES,), ND, jnp.int32)
    src = jnp.concatenate([edge_index[0], pad])
    dst = jnp.concatenate([edge_index[1], pad])
    core_off = (jnp.arange(NW, dtype=jnp.int32) % NC) * ND_P
    src5 = src.reshape(NW, NCH, CH) + core_off[:, None, None]
    dst5 = dst.reshape(NW, NCH, CH)
    ridp = jnp.concatenate(
        [res_n_id, jnp.zeros((ND_P - ND,), jnp.int32)])
    aggf, histd, xr, _ = _sc_stage(x, src5, dst5, ridp)
    agg = aggf.reshape(NC, ND_P, D)[:, :ND, :]
    histd = histd[:ND].reshape(ND, 1)
    xr = xr[:ND]
    b2 = b.reshape(1, D)
    return _tc_stage(agg, histd, xr, W, b2)


# speedup vs baseline: 11.2420x; 1.0903x over previous
import jax
import jax.numpy as jnp
from jax import lax
from jax.experimental import pallas as pl
from jax.experimental.pallas import tpu as pltpu
from jax.experimental.pallas import tpu_sc as plsc

N_NODES = 10000
N_EDGES = 320000
ND = 5000
ND_P = 5120
D = 128
NC, NS = 2, 16
NW = NC * NS
CH = 128
NCH = 80
E_P = NW * NCH * CH
EPS = E_P // NS
RPT = ND_P // NS
RSUB = 16
XR_PT = ND_P // NW
HCH = 2048
CHX = 80


def _rsqrt16(d):
    i = plsc.bitcast(d, jnp.int32)
    i = jnp.int32(0x5F3759DF) - lax.shift_right_logical(i, 1)
    y = plsc.bitcast(i, jnp.float32)
    for _ in range(3):
        y = y * (1.5 - 0.5 * d * y * y)
    return y


def _sc_body(x_hbm, src5_hbm, dst5_hbm, rid_hbm,
             agg_out, histd_out, xr_out, xsb_out,
             hidx_v, hsrc_v, hdst_v, s5_v, d5_v,
             degs_v, degd_v, a_v, hred_v, rows_v, rows2_v, sbuf_v, rid_v,
             shist_s, agg_s, sem, gsem0, gsem1, ssem0, ssem1):
    c = lax.axis_index("c")
    s = lax.axis_index("s")
    w = s * NC + c

    ones = jnp.full((16,), 1.0, jnp.float32)
    zeros = jnp.zeros((16,), jnp.float32)

    def zero_hist(i, _):
        hsrc_v[pl.ds(i * 16, 16)] = zeros
        hsrc_v[pl.ds(ND_P + i * 16, 16)] = zeros
        hdst_v[pl.ds(i * 16, 16)] = zeros
        return 0
    lax.fori_loop(0, ND_P // 16, zero_hist, 0)

    def hist_pass(idx_hbm, hist_ref):
        for cc in range(NC):
            wb = s * NC + cc

            def outer(h, _):
                pltpu.sync_copy(idx_hbm.at[wb, pl.ds(h * 16, 16)], hidx_v)

                def inner(r, _):
                    for g in range(CH // 16):
                        si = hidx_v[r, pl.ds(g * 16, 16)]
                        plsc.addupdate_scatter(hist_ref, [si], ones)
                    return 0
                lax.fori_loop(0, 16, inner, 0)
                return 0
            lax.fori_loop(0, NCH // 16, outer, 0)

    hist_pass(src5_hbm, hsrc_v)
    hist_pass(dst5_hbm, hdst_v)

    def fold_body(k, _):
        sl = pl.ds(k * 16, 16)
        hsrc_v[sl] = hsrc_v[sl] + hsrc_v[pl.ds(ND_P + k * 16, 16)]
        return 0
    lax.fori_loop(0, ND_P // 16, fold_body, 0)

    pltpu.sync_copy(hsrc_v.at[pl.ds(0, ND_P)],
                    shist_s.at[pl.ds(s * ND_P, ND_P)])
    pltpu.sync_copy(hdst_v, shist_s.at[pl.ds((NS + s) * ND_P, ND_P)])
    plsc.subcore_barrier()

    r0 = s * RPT

    def reduce_hist(which, out_ref):
        for t in range(NS):
            pltpu.sync_copy(
                shist_s.at[pl.ds((which * NS + t) * ND_P + r0, RPT)],
                hred_v.at[pl.ds(t * RPT, RPT)])

        def red_body(k, _):
            acc = hred_v[pl.ds(k * 16, 16)]
            for t in range(1, NS):
                acc = acc + hred_v[pl.ds(t * RPT + k * 16, 16)]
            out_ref[pl.ds(k * 16, 16)] = acc
            return 0
        lax.fori_loop(0, RPT // 16, red_body, 0)

    reduce_hist(0, degs_v)
    reduce_hist(1, degd_v)

    def newton_body(k, _):
        d = degs_v[pl.ds(k * 16, 16)] + 1.0
        a_v[pl.ds(k * 16, 16)] = _rsqrt16(d)
        return 0
    lax.fori_loop(0, RPT // 16, newton_body, 0)

    @pl.when(c == 0)
    def _():
        pltpu.sync_copy(degd_v, histd_out.at[pl.ds(r0, RPT)])

    def xs_body(j, _):
        rr = r0 + j * RSUB
        pltpu.sync_copy(x_hbm.at[pl.ds(rr, RSUB)], sbuf_v)
        a16 = a_v[pl.ds(j * RSUB, RSUB)]
        for r in range(RSUB):
            av = a16[r]
            for cc in range(D // 16):
                sl = pl.ds(cc * 16, 16)
                sbuf_v[r, sl] = sbuf_v[r, sl] * av
        pltpu.sync_copy(sbuf_v, xsb_out.at[pl.ds(c * ND_P + rr, RSUB)])
        return 0
    lax.fori_loop(0, RPT // RSUB, xs_body, 0)

    def zero_rows(r, _):
        for cc in range(D // 16):
            rows_v[r, pl.ds(cc * 16, 16)] = zeros
        return 0
    lax.fori_loop(0, CH, zero_rows, 0)
    for k in range(RPT // CHX):
        pltpu.sync_copy(rows_v.at[pl.ds(0, CHX)],
                        agg_s.at[pl.ds(r0 + k * CHX, CHX)])
    plsc.subcore_barrier()

    pltpu.sync_copy(src5_hbm.at[w], s5_v)
    pltpu.sync_copy(dst5_hbm.at[w], d5_v)

    def g_issue(j, buf, gsem):
        pltpu.async_copy(xsb_out.at[s5_v.at[j]], buf, gsem)

    def g_wait(j, buf, gsem):
        pltpu.make_async_copy(xsb_out.at[s5_v.at[j]], buf, gsem).wait()

    def s_issue(j, buf, ssem):
        pltpu.async_copy(buf, agg_s.at[d5_v.at[j]], ssem, add=True)

    def s_wait(j, buf, ssem):
        pltpu.make_async_copy(buf, agg_s.at[d5_v.at[j]], ssem).wait()

    g_issue(0, rows_v, gsem0)
    g_issue(1, rows2_v, gsem1)

    def edge_body(jj, _):
        j0 = 2 * jj
        j1 = j0 + 1
        g_wait(j0, rows_v, gsem0)
        s_issue(j0, rows_v, ssem0)
        g_wait(j1, rows2_v, gsem1)
        s_issue(j1, rows2_v, ssem1)
        s_wait(j0, rows_v, ssem0)

        @pl.when(jj < NCH // 2 - 1)
        def _():
            g_issue(j0 + 2, rows_v, gsem0)
        s_wait(j1, rows2_v, ssem1)

        @pl.when(jj < NCH // 2 - 1)
        def _():
            g_issue(j1 + 2, rows2_v, gsem1)
        return 0
    lax.fori_loop(0, NCH // 2, edge_body, 0)
    plsc.subcore_barrier()

    for k in range(RPT // CHX):
        rr = r0 + k * CHX
        pltpu.sync_copy(agg_s.at[pl.ds(rr, CHX)], rows_v.at[pl.ds(0, CHX)])
        pltpu.sync_copy(rows_v.at[pl.ds(0, CHX)],
                        agg_out.at[pl.ds(c * ND_P + rr, CHX)])

    for k in range(XR_PT // CHX):
        rr = w * XR_PT + k * CHX
        pltpu.sync_copy(rid_hbm.at[pl.ds(rr, CHX)], rid_v)
        pltpu.async_copy(x_hbm.at[rid_v], rows_v.at[pl.ds(0, CHX)],
                         sem).wait()
        pltpu.sync_copy(rows_v.at[pl.ds(0, CHX)], xr_out.at[pl.ds(rr, CHX)])


@jax.jit
def _sc_stage(x, src5, dst5, ridp):
    mesh = plsc.VectorSubcoreMesh(core_axis_name="c", subcore_axis_name="s")
    f = pl.kernel(
        _sc_body,
        out_type=[
            jax.ShapeDtypeStruct((NC * ND_P, D), jnp.float32),
            jax.ShapeDtypeStruct((ND_P,), jnp.float32),
            jax.ShapeDtypeStruct((ND_P, D), jnp.float32),
            jax.ShapeDtypeStruct((NC * ND_P, D), jnp.float32),
        ],
        mesh=mesh,
        scratch_types=[
            pltpu.VMEM((16, CH), jnp.int32),
            pltpu.VMEM((2 * ND_P,), jnp.float32),
            pltpu.VMEM((ND_P,), jnp.float32),
            pltpu.VMEM((NCH, CH), jnp.int32),
            pltpu.VMEM((NCH, CH), jnp.int32),
            pltpu.VMEM((RPT,), jnp.float32),
            pltpu.VMEM((RPT,), jnp.float32),
            pltpu.VMEM((RPT,), jnp.float32),
            pltpu.VMEM((NS * RPT,), jnp.float32),
            pltpu.VMEM((CH, D), jnp.float32),
            pltpu.VMEM((CH, D), jnp.float32),
            pltpu.VMEM((RSUB, D), jnp.float32),
            pltpu.VMEM((CHX,), jnp.int32),
            pltpu.VMEM_SHARED((2 * NS * ND_P,), jnp.float32),
            pltpu.VMEM_SHARED((ND_P, D), jnp.float32),
            pltpu.SemaphoreType.DMA,
            pltpu.SemaphoreType.DMA,
            pltpu.SemaphoreType.DMA,
            pltpu.SemaphoreType.DMA,
            pltpu.SemaphoreType.DMA,
        ],
        compiler_params=pltpu.CompilerParams(needs_layout_passes=False),
    )
    return f(x, src5, dst5, ridp)


def _tc_body(agg_ref, hist_ref, xr_ref, w_ref, b_ref, o_ref):
    deg = hist_ref[...] + 1.0
    agg = agg_ref[0] + agg_ref[1]
    z = agg * lax.rsqrt(deg) + xr_ref[...] * (1.0 / deg)
    out = jnp.dot(z, w_ref[...], preferred_element_type=jnp.float32)
    out = out + b_ref[...]
    m = jnp.max(out, axis=1, keepdims=True)
    ex = jnp.exp(out - m)
    lse = jnp.log(jnp.sum(ex, axis=1, keepdims=True)) + m
    o_ref[...] = out - lse


BR = 1000


@jax.jit
def _tc_stage(agg, histd, xr, W, b2):
    return pl.pallas_call(
        _tc_body,
        grid=(ND // BR,),
        in_specs=[
            pl.BlockSpec((NC, BR, D), lambda i: (0, i, 0)),
            pl.BlockSpec((BR, 1), lambda i: (i, 0)),
            pl.BlockSpec((BR, D), lambda i: (i, 0)),
            pl.BlockSpec((D, D), lambda i: (0, 0)),
            pl.BlockSpec((1, D), lambda i: (0, 0)),
        ],
        out_specs=pl.BlockSpec((BR, D), lambda i: (i, 0)),
        out_shape=jax.ShapeDtypeStruct((ND, D), jnp.float32),
    )(agg, histd, xr, W, b2)


def kernel(x, edge_index, res_n_id, W, b):
    pad = jnp.full((E_P - N_EDGES,), ND, jnp.int32)
    src = jnp.concatenate([edge_index[0], pad])
    dst = jnp.concatenate([edge_index[1], pad])
    core_off = (jnp.arange(NW, dtype=jnp.int32) % NC) * ND_P
    src5 = src.reshape(NW, NCH, CH) + core_off[:, None, None]
    dst5 = dst.reshape(NW, NCH, CH)
    ridp = jnp.concatenate(
        [res_n_id, jnp.zeros((ND_P - ND,), jnp.int32)])
    aggf, histd, xr, _ = _sc_stage(x, src5, dst5, ridp)
    agg = aggf.reshape(NC, ND_P, D)[:, :ND, :]
    histd = histd[:ND].reshape(ND, 1)
    xr = xr[:ND]
    b2 = b.reshape(1, D)
    return _tc_stage(agg, histd, xr, W, b2)


# speedup vs baseline: 22.5475x; 2.0057x over previous
import jax
import jax.numpy as jnp
from jax import lax
from jax.experimental import pallas as pl
from jax.experimental.pallas import tpu as pltpu
from jax.experimental.pallas import tpu_sc as plsc

N_NODES = 10000
N_EDGES = 320000
ND = 5000
ND_P = 5120
D = 128
NC, NS = 2, 16
NW = NC * NS
CH = 128
NCH = 80
GRP = 16
E_P = NW * NCH * CH
EPS = E_P // NS
RPT = ND_P // NS
RSUB = 16
XR_PT = ND_P // NW
CHX = 80


def _rsqrt16(d):
    i = plsc.bitcast(d, jnp.int32)
    i = jnp.int32(0x5F3759DF) - lax.shift_right_logical(i, 1)
    y = plsc.bitcast(i, jnp.float32)
    for _ in range(3):
        y = y * (1.5 - 0.5 * d * y * y)
    return y


def _sc_body(x_hbm, src5_hbm, dst5_hbm, rid_hbm,
             agg_out, histd_out, xr_out,
             a_v, shist_s, xs_s, agg_s, sem):
    c = lax.axis_index("c")
    s = lax.axis_index("s")
    w = s * NC + c
    r0 = s * RPT

    ones = jnp.full((16,), 1.0, jnp.float32)
    zeros = jnp.zeros((16,), jnp.float32)

    def hist_phase(hidx_v, hsrc_v, hdst_v, hred_v, degs_v, degd_v):
        def zero_hist(i, _):
            hsrc_v[pl.ds(i * 16, 16)] = zeros
            hdst_v[pl.ds(i * 16, 16)] = zeros
            return 0
        lax.fori_loop(0, ND_P // 16, zero_hist, 0)

        def hist_pass(idx_hbm, hist_ref):
            for cc in range(NC):
                wb = s * NC + cc

                def outer(h, _):
                    pltpu.sync_copy(idx_hbm.at[wb, pl.ds(h * 16, 16)],
                                    hidx_v)

                    def inner(r, _):
                        for g in range(CH // 16):
                            si = hidx_v[r, pl.ds(g * 16, 16)]
                            plsc.addupdate_scatter(hist_ref, [si], ones)
                        return 0
                    lax.fori_loop(0, 16, inner, 0)
                    return 0
                lax.fori_loop(0, NCH // 16, outer, 0)

        hist_pass(src5_hbm, hsrc_v)
        hist_pass(dst5_hbm, hdst_v)

        pltpu.sync_copy(hsrc_v, shist_s.at[pl.ds(s * ND_P, ND_P)])
        pltpu.sync_copy(hdst_v, shist_s.at[pl.ds((NS + s) * ND_P, ND_P)])
        plsc.subcore_barrier()

        def reduce_hist(which, out_ref):
            for t in range(NS):
                pltpu.sync_copy(
                    shist_s.at[pl.ds((which * NS + t) * ND_P + r0, RPT)],
                    hred_v.at[pl.ds(t * RPT, RPT)])

            def red_body(k, _):
                acc = hred_v[pl.ds(k * 16, 16)]
                for t in range(1, NS):
                    acc = acc + hred_v[pl.ds(t * RPT + k * 16, 16)]
                out_ref[pl.ds(k * 16, 16)] = acc
                return 0
            lax.fori_loop(0, RPT // 16, red_body, 0)

        reduce_hist(0, degs_v)
        reduce_hist(1, degd_v)

        def newton_body(k, _):
            d = degs_v[pl.ds(k * 16, 16)] + 1.0
            a_v[pl.ds(k * 16, 16)] = _rsqrt16(d)
            return 0
        lax.fori_loop(0, RPT // 16, newton_body, 0)

        @pl.when(c == 0)
        def _():
            pltpu.sync_copy(degd_v, histd_out.at[pl.ds(r0, RPT)])

    pl.run_scoped(
        hist_phase,
        pltpu.VMEM((16, CH), jnp.int32),
        pltpu.VMEM((ND_P,), jnp.float32),
        pltpu.VMEM((ND_P,), jnp.float32),
        pltpu.VMEM((NS * RPT,), jnp.float32),
        pltpu.VMEM((RPT,), jnp.float32),
        pltpu.VMEM((RPT,), jnp.float32),
    )

    def xs_phase(sbuf_v, zbuf_v):
        def xs_body(j, _):
            rr = r0 + j * RSUB
            pltpu.sync_copy(x_hbm.at[pl.ds(rr, RSUB)], sbuf_v)
            a16 = a_v[pl.ds(j * RSUB, RSUB)]
            for r in range(RSUB):
                av = a16[r]
                for cc in range(D // 16):
                    sl = pl.ds(cc * 16, 16)
                    sbuf_v[r, sl] = sbuf_v[r, sl] * av
            pltpu.sync_copy(sbuf_v, xs_s.at[pl.ds(rr, RSUB)])
            return 0
        lax.fori_loop(0, RPT // RSUB, xs_body, 0)

        def zero_rows(r, _):
            for cc in range(D // 16):
                zbuf_v[r, pl.ds(cc * 16, 16)] = zeros
            return 0
        lax.fori_loop(0, CHX, zero_rows, 0)
        for k in range(RPT // CHX):
            pltpu.sync_copy(zbuf_v, agg_s.at[pl.ds(r0 + k * CHX, CHX)])

    pl.run_scoped(
        xs_phase,
        pltpu.VMEM((RSUB, D), jnp.float32),
        pltpu.VMEM((CHX, D), jnp.float32),
    )
    plsc.subcore_barrier()

    def edge_phase(s5_v, d5_v, rows_v, rows2_v, gsem0, gsem1, ssem0, ssem1):
        def g_issue(j, buf, gsem):
            pltpu.async_copy(xs_s.at[s5_v.at[j]], buf, gsem)

        def g_wait(j, buf, gsem):
            pltpu.make_async_copy(xs_s.at[s5_v.at[j]], buf, gsem).wait()

        def s_issue(j, buf, ssem):
            pltpu.async_copy(buf, agg_s.at[d5_v.at[j]], ssem, add=True)

        def s_wait(j, buf, ssem):
            pltpu.make_async_copy(buf, agg_s.at[d5_v.at[j]], ssem).wait()

        def group(g, _):
            pltpu.sync_copy(src5_hbm.at[w, pl.ds(g * GRP, GRP)], s5_v)
            pltpu.sync_copy(dst5_hbm.at[w, pl.ds(g * GRP, GRP)], d5_v)
            g_issue(0, rows_v, gsem0)
            g_issue(1, rows2_v, gsem1)

            def pair(jj, _):
                j0 = 2 * jj
                j1 = j0 + 1
                g_wait(j0, rows_v, gsem0)
                s_issue(j0, rows_v, ssem0)
                g_wait(j1, rows2_v, gsem1)
                s_wait(j0, rows_v, ssem0)
                s_issue(j1, rows2_v, ssem1)

                @pl.when(jj < GRP // 2 - 1)
                def _():
                    g_issue(j0 + 2, rows_v, gsem0)
                s_wait(j1, rows2_v, ssem1)

                @pl.when(jj < GRP // 2 - 1)
                def _():
                    g_issue(j1 + 2, rows2_v, gsem1)
                return 0
            lax.fori_loop(0, GRP // 2, pair, 0)
            return 0
        lax.fori_loop(0, NCH // GRP, group, 0)

    pl.run_scoped(
        edge_phase,
        pltpu.VMEM((GRP, CH), jnp.int32),
        pltpu.VMEM((GRP, CH), jnp.int32),
        pltpu.VMEM((CH, D), jnp.float32),
        pltpu.VMEM((CH, D), jnp.float32),
        pltpu.SemaphoreType.DMA,
        pltpu.SemaphoreType.DMA,
        pltpu.SemaphoreType.DMA,
        pltpu.SemaphoreType.DMA,
    )
    plsc.subcore_barrier()

    def export_phase(ebuf_v, rid_v):
        for k in range(RPT // CHX):
            rr = r0 + k * CHX
            pltpu.sync_copy(agg_s.at[pl.ds(rr, CHX)], ebuf_v)
            pltpu.sync_copy(ebuf_v, agg_out.at[pl.ds(c * ND_P + rr, CHX)])

        for k in range(XR_PT // CHX):
            rr = w * XR_PT + k * CHX
            pltpu.sync_copy(rid_hbm.at[pl.ds(rr, CHX)], rid_v)
            pltpu.async_copy(x_hbm.at[rid_v], ebuf_v, sem).wait()
            pltpu.sync_copy(ebuf_v, xr_out.at[pl.ds(rr, CHX)])

    pl.run_scoped(
        export_phase,
        pltpu.VMEM((CHX, D), jnp.float32),
        pltpu.VMEM((CHX,), jnp.int32),
    )


@jax.jit
def _sc_stage(x, src5, dst5, ridp):
    mesh = plsc.VectorSubcoreMesh(core_axis_name="c", subcore_axis_name="s")
    f = pl.kernel(
        _sc_body,
        out_type=[
            jax.ShapeDtypeStruct((NC * ND_P, D), jnp.float32),
            jax.ShapeDtypeStruct((ND_P,), jnp.float32),
            jax.ShapeDtypeStruct((ND_P, D), jnp.float32),
        ],
        mesh=mesh,
        scratch_types=[
            pltpu.VMEM((RPT,), jnp.float32),
            pltpu.VMEM_SHARED((2 * NS * ND_P,), jnp.float32),
            pltpu.VMEM_SHARED((ND_P, D), jnp.float32),
            pltpu.VMEM_SHARED((ND_P, D), jnp.float32),
            pltpu.SemaphoreType.DMA,
        ],
        compiler_params=pltpu.CompilerParams(needs_layout_passes=False),
    )
    return f(x, src5, dst5, ridp)


def _tc_body(agg_ref, hist_ref, xr_ref, w_ref, b_ref, o_ref):
    deg = hist_ref[...] + 1.0
    agg = agg_ref[0] + agg_ref[1]
    z = agg * lax.rsqrt(deg) + xr_ref[...] * (1.0 / deg)
    out = jnp.dot(z, w_ref[...], preferred_element_type=jnp.float32)
    out = out + b_ref[...]
    m = jnp.max(out, axis=1, keepdims=True)
    ex = jnp.exp(out - m)
    lse = jnp.log(jnp.sum(ex, axis=1, keepdims=True)) + m
    o_ref[...] = out - lse


BR = 1000


@jax.jit
def _tc_stage(agg, histd, xr, W, b2):
    return pl.pallas_call(
        _tc_body,
        grid=(ND // BR,),
        in_specs=[
            pl.BlockSpec((NC, BR, D), lambda i: (0, i, 0)),
            pl.BlockSpec((BR, 1), lambda i: (i, 0)),
            pl.BlockSpec((BR, D), lambda i: (i, 0)),
            pl.BlockSpec((D, D), lambda i: (0, 0)),
            pl.BlockSpec((1, D), lambda i: (0, 0)),
        ],
        out_specs=pl.BlockSpec((BR, D), lambda i: (i, 0)),
        out_shape=jax.ShapeDtypeStruct((ND, D), jnp.float32),
    )(agg, histd, xr, W, b2)


def kernel(x, edge_index, res_n_id, W, b):
    pad = jnp.full((E_P - N_EDGES,), ND, jnp.int32)
    src = jnp.concatenate([edge_index[0], pad])
    dst = jnp.concatenate([edge_index[1], pad])
    src5 = src.reshape(NW, NCH, CH)
    dst5 = dst.reshape(NW, NCH, CH)
    ridp = jnp.concatenate(
        [res_n_id, jnp.zeros((ND_P - ND,), jnp.int32)])
    aggf, histd, xr = _sc_stage(x, src5, dst5, ridp)
    agg = aggf.reshape(NC, ND_P, D)[:, :ND, :]
    histd = histd[:ND].reshape(ND, 1)
    xr = xr[:ND]
    b2 = b.reshape(1, D)
    return _tc_stage(agg, histd, xr, W, b2)


# speedup vs baseline: 23.4581x; 1.0404x over previous
import jax
import jax.numpy as jnp
from jax import lax
from jax.experimental import pallas as pl
from jax.experimental.pallas import tpu as pltpu
from jax.experimental.pallas import tpu_sc as plsc

N_NODES = 10000
N_EDGES = 320000
ND = 5000
ND_P = 5120
D = 128
NC, NS = 2, 16
NW = NC * NS
CH = 128
NCH = 80
GRP = 16
E_P = NW * NCH * CH
EPS = E_P // NS
RPT = ND_P // NS
RSUB = 16
XR_PT = ND_P // NW
CHX = 80


def _rsqrt16(d):
    i = plsc.bitcast(d, jnp.int32)
    i = jnp.int32(0x5F3759DF) - lax.shift_right_logical(i, 1)
    y = plsc.bitcast(i, jnp.float32)
    for _ in range(3):
        y = y * (1.5 - 0.5 * d * y * y)
    return y


def _sc_body(x_hbm, src5_hbm, dst5_hbm, rid_hbm,
             agg_out, histd_out, xr_out,
             a_v, shist_s, xs_s, agg_s, sem):
    c = lax.axis_index("c")
    s = lax.axis_index("s")
    w = s * NC + c
    r0 = s * RPT

    ones = jnp.full((16,), 1.0, jnp.float32)
    zeros = jnp.zeros((16,), jnp.float32)

    def hist_phase(hidx_v, hidx2_v, hsrc_v, hdst_v, hred_v, degs_v, degd_v,
                   hsem0, hsem1):
        def zero_hist(i, _):
            hsrc_v[pl.ds(i * 16, 16)] = zeros
            hdst_v[pl.ds(i * 16, 16)] = zeros
            return 0
        lax.fori_loop(0, ND_P // 16, zero_hist, 0)

        NB = NC * (NCH // 16)

        def hist_pass(idx_hbm, hist_ref):
            def blk_ref(b):
                return idx_hbm.at[s * NC + b // (NCH // 16),
                                  pl.ds((b % (NCH // 16)) * 16, 16)]

            def consume(buf):
                def inner(r, _):
                    for g in range(CH // 16):
                        si = buf[r, pl.ds(g * 16, 16)]
                        plsc.addupdate_scatter(hist_ref, [si], ones)
                    return 0
                lax.fori_loop(0, 16, inner, 0)

            pltpu.async_copy(blk_ref(0), hidx_v, hsem0)

            def pair(p, _):
                b0 = 2 * p
                b1 = b0 + 1
                pltpu.make_async_copy(blk_ref(b0), hidx_v, hsem0).wait()
                pltpu.async_copy(blk_ref(b1), hidx2_v, hsem1)
                consume(hidx_v)
                pltpu.make_async_copy(blk_ref(b1), hidx2_v, hsem1).wait()

                @pl.when(p < NB // 2 - 1)
                def _():
                    pltpu.async_copy(blk_ref(b0 + 2), hidx_v, hsem0)
                consume(hidx2_v)
                return 0
            lax.fori_loop(0, NB // 2, pair, 0)

        hist_pass(src5_hbm, hsrc_v)
        hist_pass(dst5_hbm, hdst_v)

        pltpu.sync_copy(hsrc_v, shist_s.at[pl.ds(s * ND_P, ND_P)])
        pltpu.sync_copy(hdst_v, shist_s.at[pl.ds((NS + s) * ND_P, ND_P)])
        plsc.subcore_barrier()

        def reduce_hist(which, out_ref):
            for t in range(NS):
                pltpu.sync_copy(
                    shist_s.at[pl.ds((which * NS + t) * ND_P + r0, RPT)],
                    hred_v.at[pl.ds(t * RPT, RPT)])

            def red_body(k, _):
                acc = hred_v[pl.ds(k * 16, 16)]
                for t in range(1, NS):
                    acc = acc + hred_v[pl.ds(t * RPT + k * 16, 16)]
                out_ref[pl.ds(k * 16, 16)] = acc
                return 0
            lax.fori_loop(0, RPT // 16, red_body, 0)

        reduce_hist(0, degs_v)
        reduce_hist(1, degd_v)

        def newton_body(k, _):
            d = degs_v[pl.ds(k * 16, 16)] + 1.0
            a_v[pl.ds(k * 16, 16)] = _rsqrt16(d)
            return 0
        lax.fori_loop(0, RPT // 16, newton_body, 0)

        @pl.when(c == 0)
        def _():
            pltpu.sync_copy(degd_v, histd_out.at[pl.ds(r0, RPT)])

    pl.run_scoped(
        hist_phase,
        pltpu.VMEM((16, CH), jnp.int32),
        pltpu.VMEM((16, CH), jnp.int32),
        pltpu.VMEM((ND_P,), jnp.float32),
        pltpu.VMEM((ND_P,), jnp.float32),
        pltpu.VMEM((NS * RPT,), jnp.float32),
        pltpu.VMEM((RPT,), jnp.float32),
        pltpu.VMEM((RPT,), jnp.float32),
        pltpu.SemaphoreType.DMA,
        pltpu.SemaphoreType.DMA,
    )

    def xs_phase(sbuf_v, sbuf2_v, zbuf_v, xsem0, xsem1):
        def xrow_ref(j):
            return x_hbm.at[pl.ds(r0 + j * RSUB, RSUB)]

        def scale_store(j, buf):
            a16 = a_v[pl.ds(j * RSUB, RSUB)]
            for r in range(RSUB):
                av = a16[r]
                for cc in range(D // 16):
                    sl = pl.ds(cc * 16, 16)
                    buf[r, sl] = buf[r, sl] * av
            pltpu.sync_copy(buf, xs_s.at[pl.ds(r0 + j * RSUB, RSUB)])

        NXB = RPT // RSUB
        pltpu.async_copy(xrow_ref(0), sbuf_v, xsem0)

        def xs_pair(p, _):
            j0 = 2 * p
            j1 = j0 + 1
            pltpu.make_async_copy(xrow_ref(j0), sbuf_v, xsem0).wait()
            pltpu.async_copy(xrow_ref(j1), sbuf2_v, xsem1)
            scale_store(j0, sbuf_v)
            pltpu.make_async_copy(xrow_ref(j1), sbuf2_v, xsem1).wait()

            @pl.when(p < NXB // 2 - 1)
            def _():
                pltpu.async_copy(xrow_ref(j0 + 2), sbuf_v, xsem0)
            scale_store(j1, sbuf2_v)
            return 0
        lax.fori_loop(0, NXB // 2, xs_pair, 0)

        def zero_rows(r, _):
            for cc in range(D // 16):
                zbuf_v[r, pl.ds(cc * 16, 16)] = zeros
            return 0
        lax.fori_loop(0, CHX, zero_rows, 0)
        for k in range(RPT // CHX):
            pltpu.sync_copy(zbuf_v, agg_s.at[pl.ds(r0 + k * CHX, CHX)])

    pl.run_scoped(
        xs_phase,
        pltpu.VMEM((RSUB, D), jnp.float32),
        pltpu.VMEM((RSUB, D), jnp.float32),
        pltpu.VMEM((CHX, D), jnp.float32),
        pltpu.SemaphoreType.DMA,
        pltpu.SemaphoreType.DMA,
    )
    plsc.subcore_barrier()

    def edge_phase(s5_v, d5_v, rows_v, rows2_v, gsem0, gsem1, ssem0, ssem1):
        def g_issue(j, buf, gsem):
            pltpu.async_copy(xs_s.at[s5_v.at[j]], buf, gsem)

        def g_wait(j, buf, gsem):
            pltpu.make_async_copy(xs_s.at[s5_v.at[j]], buf, gsem).wait()

        def s_issue(j, buf, ssem):
            pltpu.async_copy(buf, agg_s.at[d5_v.at[j]], ssem, add=True)

        def s_wait(j, buf, ssem):
            pltpu.make_async_copy(buf, agg_s.at[d5_v.at[j]], ssem).wait()

        def group(g, _):
            pltpu.sync_copy(src5_hbm.at[w, pl.ds(g * GRP, GRP)], s5_v)
            pltpu.sync_copy(dst5_hbm.at[w, pl.ds(g * GRP, GRP)], d5_v)
            g_issue(0, rows_v, gsem0)
            g_issue(1, rows2_v, gsem1)

            def pair(jj, _):
                j0 = 2 * jj
                j1 = j0 + 1
                g_wait(j0, rows_v, gsem0)
                s_issue(j0, rows_v, ssem0)
                g_wait(j1, rows2_v, gsem1)
                s_wait(j0, rows_v, ssem0)
                s_issue(j1, rows2_v, ssem1)

                @pl.when(jj < GRP // 2 - 1)
                def _():
                    g_issue(j0 + 2, rows_v, gsem0)
                s_wait(j1, rows2_v, ssem1)

                @pl.when(jj < GRP // 2 - 1)
                def _():
                    g_issue(j1 + 2, rows2_v, gsem1)
                return 0
            lax.fori_loop(0, GRP // 2, pair, 0)
            return 0
        lax.fori_loop(0, NCH // GRP, group, 0)

    pl.run_scoped(
        edge_phase,
        pltpu.VMEM((GRP, CH), jnp.int32),
        pltpu.VMEM((GRP, CH), jnp.int32),
        pltpu.VMEM((CH, D), jnp.float32),
        pltpu.VMEM((CH, D), jnp.float32),
        pltpu.SemaphoreType.DMA,
        pltpu.SemaphoreType.DMA,
        pltpu.SemaphoreType.DMA,
        pltpu.SemaphoreType.DMA,
    )
    plsc.subcore_barrier()

    def export_phase(ebuf_v, rid_v):
        for k in range(RPT // CHX):
            rr = r0 + k * CHX
            pltpu.sync_copy(agg_s.at[pl.ds(rr, CHX)],
                            agg_out.at[pl.ds(c * ND_P + rr, CHX)])

        for k in range(XR_PT // CHX):
            rr = w * XR_PT + k * CHX
            pltpu.sync_copy(rid_hbm.at[pl.ds(rr, CHX)], rid_v)
            pltpu.async_copy(x_hbm.at[rid_v], ebuf_v, sem).wait()
            pltpu.sync_copy(ebuf_v, xr_out.at[pl.ds(rr, CHX)])

    pl.run_scoped(
        export_phase,
        pltpu.VMEM((CHX, D), jnp.float32),
        pltpu.VMEM((CHX,), jnp.int32),
    )


@jax.jit
def _sc_stage(x, src5, dst5, ridp):
    mesh = plsc.VectorSubcoreMesh(core_axis_name="c", subcore_axis_name="s")
    f = pl.kernel(
        _sc_body,
        out_type=[
            jax.ShapeDtypeStruct((NC * ND_P, D), jnp.float32),
            jax.ShapeDtypeStruct((ND_P,), jnp.float32),
            jax.ShapeDtypeStruct((ND_P, D), jnp.float32),
        ],
        mesh=mesh,
        scratch_types=[
            pltpu.VMEM((RPT,), jnp.float32),
            pltpu.VMEM_SHARED((2 * NS * ND_P,), jnp.float32),
            pltpu.VMEM_SHARED((ND_P, D), jnp.float32),
            pltpu.VMEM_SHARED((ND_P, D), jnp.float32),
            pltpu.SemaphoreType.DMA,
        ],
        compiler_params=pltpu.CompilerParams(needs_layout_passes=False),
    )
    return f(x, src5, dst5, ridp)


def _tc_body(agg_ref, hist_ref, xr_ref, w_ref, b_ref, o_ref):
    deg = hist_ref[...] + 1.0
    agg = agg_ref[0] + agg_ref[1]
    z = agg * lax.rsqrt(deg) + xr_ref[...] * (1.0 / deg)
    out = jnp.dot(z, w_ref[...], preferred_element_type=jnp.float32)
    out = out + b_ref[...]
    m = jnp.max(out, axis=1, keepdims=True)
    ex = jnp.exp(out - m)
    lse = jnp.log(jnp.sum(ex, axis=1, keepdims=True)) + m
    o_ref[...] = out - lse


BR = 1000


@jax.jit
def _tc_stage(agg, histd, xr, W, b2):
    return pl.pallas_call(
        _tc_body,
        grid=(ND // BR,),
        in_specs=[
            pl.BlockSpec((NC, BR, D), lambda i: (0, i, 0)),
            pl.BlockSpec((BR, 1), lambda i: (i, 0)),
            pl.BlockSpec((BR, D), lambda i: (i, 0)),
            pl.BlockSpec((D, D), lambda i: (0, 0)),
            pl.BlockSpec((1, D), lambda i: (0, 0)),
        ],
        out_specs=pl.BlockSpec((BR, D), lambda i: (i, 0)),
        out_shape=jax.ShapeDtypeStruct((ND, D), jnp.float32),
    )(agg, histd, xr, W, b2)


def kernel(x, edge_index, res_n_id, W, b):
    pad = jnp.full((E_P - N_EDGES,), ND, jnp.int32)
    src = jnp.concatenate([edge_index[0], pad])
    dst = jnp.concatenate([edge_index[1], pad])
    src5 = src.reshape(NW, NCH, CH)
    dst5 = dst.reshape(NW, NCH, CH)
    ridp = jnp.concatenate(
        [res_n_id, jnp.zeros((ND_P - ND,), jnp.int32)])
    aggf, histd, xr = _sc_stage(x, src5, dst5, ridp)
    agg = aggf.reshape(NC, ND_P, D)[:, :ND, :]
    histd = histd[:ND].reshape(ND, 1)
    xr = xr[:ND]
    b2 = b.reshape(1, D)
    return _tc_stage(agg, histd, xr, W, b2)
